# baseline (speedup 1.0000x reference)
import functools

import jax
import jax.numpy as jnp
from jax import lax
from jax.experimental import pallas as pl
from jax.experimental.pallas import tpu as pltpu
from jax.experimental.pallas import tpu_sc as plsc

NUM_EMB = 120
EMB_DIM = 64
CHUNK = 128
CHUNKS_PER_SLAB = 4
SLAB = CHUNK * CHUNKS_PER_SLAB


@functools.lru_cache(maxsize=None)
def _build_sc_gather(n_slabs: int):
    info = plsc.get_sparse_core_info()
    num_cores = info.num_cores
    num_workers = info.num_cores * info.num_subcores
    per_w = n_slabs // num_workers

    mesh = plsc.VectorSubcoreMesh(core_axis_name="c", subcore_axis_name="s")

    @functools.partial(
        pl.kernel,
        mesh=mesh,
        compiler_params=pltpu.CompilerParams(use_tc_tiling_on_sc=False),
        out_type=jax.ShapeDtypeStruct((n_slabs, CHUNKS_PER_SLAB, CHUNK, EMB_DIM),
                                      jnp.float32),
        scratch_types=[
            pltpu.VMEM((2, CHUNKS_PER_SLAB, CHUNK), jnp.int32),
            pltpu.VMEM((2, CHUNKS_PER_SLAB, CHUNK, EMB_DIM), jnp.float32),
            pltpu.VMEM_SHARED((NUM_EMB, EMB_DIM), jnp.float32),
            pltpu.SemaphoreType.DMA,
            pltpu.SemaphoreType.DMA,
            pltpu.SemaphoreType.DMA((2,)),
        ],
    )
    def gather_kernel(ids_hbm, table_hbm, out_hbm, idx_v, rows_v, table_v,
                      isem, gsem, ssem):
        wid = lax.axis_index("s") * num_cores + lax.axis_index("c")
        base = wid * per_w

        @pl.when(lax.axis_index("s") == 0)
        def _():
            pltpu.sync_copy(table_hbm, table_v)
        plsc.subcore_barrier()

        def body(s, carry):
            b = lax.rem(s, 2)
            s_abs = base + s

            @pl.when(s >= 2)
            def _():
                for j in range(CHUNKS_PER_SLAB):
                    pltpu.make_async_copy(
                        rows_v.at[b, j], out_hbm.at[s_abs, j],
                        ssem.at[b]).wait()

            pltpu.make_async_copy(
                ids_hbm.at[s_abs], idx_v.at[b], isem).wait()

            copies = [
                pltpu.async_copy(
                    table_v.at[idx_v.at[b, j]], rows_v.at[b, j], gsem)
                for j in range(CHUNKS_PER_SLAB)
            ]

            @pl.when(s + 1 < per_w)
            def _():
                pltpu.async_copy(
                    ids_hbm.at[s_abs + 1], idx_v.at[1 - b], isem)

            for c in copies:
                c.wait()

            for j in range(CHUNKS_PER_SLAB):
                pltpu.async_copy(
                    rows_v.at[b, j], out_hbm.at[s_abs, j], ssem.at[b])
            return carry

        pltpu.async_copy(ids_hbm.at[base], idx_v.at[0], isem)
        lax.fori_loop(0, per_w, body, 0, unroll=False)

        for b in range(2):
            for j in range(CHUNKS_PER_SLAB):
                pltpu.make_async_copy(
                    rows_v.at[b, j], out_hbm.at[base, j], ssem.at[b]).wait()

    return gather_kernel


def kernel(label_ids, table):
    B, N = label_ids.shape
    total = B * N
    assert total % SLAB == 0
    n_slabs = total // SLAB
    ids = label_ids.reshape(n_slabs, CHUNKS_PER_SLAB, CHUNK).astype(jnp.int32)
    out = _build_sc_gather(n_slabs)(ids, table)
    return out.reshape(B, N, EMB_DIM)


# speedup vs baseline: 1.1325x; 1.1325x over previous
import functools

import jax
import jax.numpy as jnp
from jax import lax
from jax.experimental import pallas as pl
from jax.experimental.pallas import tpu as pltpu
from jax.experimental.pallas import tpu_sc as plsc

NUM_EMB = 120
EMB_DIM = 64
CHUNK = 128
CHUNKS_PER_SLAB = 4
SLAB = CHUNK * CHUNKS_PER_SLAB


@functools.lru_cache(maxsize=None)
def _build_sc_gather(n_slabs: int):
    info = plsc.get_sparse_core_info()
    num_cores = info.num_cores
    num_workers = info.num_cores * info.num_subcores
    per_w = n_slabs // num_workers

    mesh = plsc.VectorSubcoreMesh(core_axis_name="c", subcore_axis_name="s")

    @functools.partial(
        pl.kernel,
        mesh=mesh,
        compiler_params=pltpu.CompilerParams(use_tc_tiling_on_sc=False),
        out_type=jax.ShapeDtypeStruct((n_slabs, CHUNKS_PER_SLAB, CHUNK, EMB_DIM),
                                      jnp.float32),
        scratch_types=[
            pltpu.VMEM((2, CHUNKS_PER_SLAB, CHUNK), jnp.int32),
            pltpu.VMEM((2, CHUNKS_PER_SLAB, CHUNK, EMB_DIM), jnp.float32),
            pltpu.VMEM_SHARED((NUM_EMB, EMB_DIM), jnp.float32),
            pltpu.SemaphoreType.DMA,
            pltpu.SemaphoreType.DMA,
            pltpu.SemaphoreType.DMA((2,)),
        ],
    )
    def gather_kernel(ids_hbm, table_hbm, out_hbm, idx_v, rows_v, table_v,
                      isem, gsem, ssem):
        wid = lax.axis_index("s") * num_cores + lax.axis_index("c")
        base = wid * per_w

        @pl.when(lax.axis_index("s") == 0)
        def _():
            pltpu.sync_copy(table_hbm, table_v)
        plsc.subcore_barrier()

        def body(s, carry):
            b = lax.rem(s, 2)
            s_abs = base + s

            @pl.when(s >= 2)
            def _():
                for j in range(CHUNKS_PER_SLAB):
                    pltpu.make_async_copy(
                        rows_v.at[b, j], out_hbm.at[s_abs, j],
                        ssem.at[b]).wait()

            pltpu.make_async_copy(
                ids_hbm.at[s_abs], idx_v.at[b], isem).wait()

            copies = [
                pltpu.async_copy(
                    table_v.at[idx_v.at[b, j]], rows_v.at[b, j], gsem)
                for j in range(CHUNKS_PER_SLAB)
            ]

            @pl.when(s + 1 < per_w)
            def _():
                pltpu.async_copy(
                    ids_hbm.at[s_abs + 1], idx_v.at[1 - b], isem)

            for c in copies:
                c.wait()

            for j in range(CHUNKS_PER_SLAB):
                pltpu.async_copy(
                    rows_v.at[b, j], out_hbm.at[s_abs, j], ssem.at[b])
            return carry

        pltpu.async_copy(ids_hbm.at[base], idx_v.at[0], isem)
        lax.fori_loop(0, per_w, body, 0, unroll=False)

        for b in range(2):
            for j in range(CHUNKS_PER_SLAB):
                pltpu.make_async_copy(
                    rows_v.at[b, j], out_hbm.at[base, j], ssem.at[b]).wait()

    return gather_kernel


TC_ROWS = 2048


@functools.lru_cache(maxsize=None)
def _build_tc_encode(n_blocks: int):
    def body(ids_ref, tab_ref, out_ref):
        ids = ids_ref[0, 0, :]
        onehot = (ids[:, None] == lax.broadcasted_iota(
            jnp.int32, (TC_ROWS, 128), 1)).astype(jnp.float32)
        out_ref[0] = jnp.dot(onehot, tab_ref[...],
                             preferred_element_type=jnp.float32)

    return pl.pallas_call(
        body,
        grid=(n_blocks,),
        in_specs=[
            pl.BlockSpec((1, 1, TC_ROWS), lambda i: (i, 0, 0)),
            pl.BlockSpec((128, EMB_DIM), lambda i: (0, 0)),
        ],
        out_specs=pl.BlockSpec((1, TC_ROWS, EMB_DIM), lambda i: (i, 0, 0)),
        out_shape=jax.ShapeDtypeStruct((n_blocks, TC_ROWS, EMB_DIM),
                                       jnp.float32),
    )


def kernel(label_ids, table):
    B, N = label_ids.shape
    total = B * N
    n_blocks = total // TC_ROWS
    ids = label_ids.reshape(n_blocks, 1, TC_ROWS).astype(jnp.int32)
    table_pad = jnp.pad(table, ((0, 128 - NUM_EMB), (0, 0)))
    out = _build_tc_encode(n_blocks)(ids, table_pad)
    return out.reshape(B, N, EMB_DIM)
